# Initial kernel scaffold; baseline (speedup 1.0000x reference)
import jax
import jax.numpy as jnp
from jax.experimental import pallas as pl


def kernel(x, edge_index, edge_type, edge_norm, weight, a_weight, w_comp, w_comp_a):
    raise NotImplementedError("write your pallas kernel here")


# speedup vs baseline: 10.3260x; 10.3260x over previous
import functools

import jax
import jax.numpy as jnp
from jax import lax
from jax.experimental import pallas as pl
from jax.experimental.pallas import tpu as pltpu
from jax.experimental.pallas import tpu_sc as plsc

N = 10000
E = 320000
FIN = 128
FOUT = 128
R = 16
NBASES = 8

NC = 2
NS = 16
NW = NC * NS

NPAD = 10240
CH = 128
EPW = 10112
NCH = EPW // CH
EPAD = EPW * NW
SLN = NPAD // NS

RB = 512


def _proj_body(x_ref, wt_ref, awt_ref, wc_ref, wca_ref, hp_ref, ep_ref):
    r = pl.program_id(1)
    xb = x_ref[...]
    wr = jnp.tensordot(wc_ref[r], wt_ref[...], axes=1)
    hp_ref[0] = jnp.dot(xb, wr, preferred_element_type=jnp.float32)
    @pl.when(r == 0)
    def _():
        awm = jnp.tensordot(wca_ref[...], awt_ref[...], axes=1)
        a = lax.dot_general(xb, awm, (((1,), (1,)), ((), ())),
                            preferred_element_type=jnp.float32)
        ep_ref[...] = jnp.exp(jnp.tanh(a))


_proj = pl.pallas_call(
    _proj_body,
    grid=(NPAD // RB, R),
    in_specs=[
        pl.BlockSpec((RB, FIN), lambda n, r: (n, 0)),
        pl.BlockSpec((NBASES, FIN, FOUT), lambda n, r: (0, 0, 0)),
        pl.BlockSpec((NBASES, FIN), lambda n, r: (0, 0)),
        pl.BlockSpec((R, NBASES), lambda n, r: (0, 0)),
        pl.BlockSpec((R, NBASES), lambda n, r: (0, 0)),
    ],
    out_specs=[
        pl.BlockSpec((1, RB, FOUT), lambda n, r: (r, n, 0)),
        pl.BlockSpec((RB, R), lambda n, r: (n, 0)),
    ],
    out_shape=[
        jax.ShapeDtypeStruct((R, NPAD, FOUT), jnp.float32),
        jax.ShapeDtypeStruct((NPAD, R), jnp.float32),
    ],
)


def _atten_body(ep_hbm, src_hbm, et_hbm, evals_hbm, asum_hbm,
                srcb, etb, gidxb, evb, zb, ep_sh, asum_sh):
    c = lax.axis_index("c")
    s = lax.axis_index("s")
    wid = s * NC + c

    esl = (R * NPAD) // NS
    pltpu.sync_copy(ep_hbm.at[pl.ds(s * esl, esl)], ep_sh.at[pl.ds(s * esl, esl)])
    for i in range(SLN // 16):
        zb[pl.ds(16 * i, 16)] = jnp.zeros((16,), jnp.float32)
    pltpu.sync_copy(zb, asum_sh.at[pl.ds(s * SLN, SLN)])
    plsc.subcore_barrier()

    ebase = wid * EPW

    def chunk(i, carry):
        b = ebase + i * CH
        pltpu.sync_copy(src_hbm.at[pl.ds(b, CH)], srcb)
        pltpu.sync_copy(et_hbm.at[pl.ds(b, CH)], etb)
        for j in range(CH // 16):
            sl = pl.ds(16 * j, 16)
            gidxb[sl] = srcb[sl] * R + etb[sl]
        pltpu.sync_copy(ep_sh.at[gidxb], evb)
        pltpu.sync_copy(evb, evals_hbm.at[pl.ds(b, CH)])
        pltpu.sync_copy(evb, asum_sh.at[srcb], add=True)
        return carry

    lax.fori_loop(0, NCH, chunk, 0)

    plsc.subcore_barrier()
    pltpu.sync_copy(asum_sh.at[pl.ds(s * SLN, SLN)],
                    asum_hbm.at[c, pl.ds(s * SLN, SLN)])


_atten = pl.kernel(
    _atten_body,
    out_type=(jax.ShapeDtypeStruct((EPAD,), jnp.float32),
              jax.ShapeDtypeStruct((NC, NPAD), jnp.float32)),
    mesh=plsc.VectorSubcoreMesh(core_axis_name="c", subcore_axis_name="s"),
    scratch_types=[
        pltpu.VMEM((CH,), jnp.int32),
        pltpu.VMEM((CH,), jnp.int32),
        pltpu.VMEM((CH,), jnp.int32),
        pltpu.VMEM((CH,), jnp.float32),
        pltpu.VMEM((SLN,), jnp.float32),
        pltpu.VMEM_SHARED((R * NPAD,), jnp.float32),
        pltpu.VMEM_SHARED((NPAD,), jnp.float32),
    ],
)


ZR = 64


def _agg_body(hp_hbm, src_hbm, et_hbm, dst_hbm, norm_hbm, evals_hbm, asum2_hbm,
              outp_hbm,
              srcb, etb, dstb, gidxb, normb, evb, coefb, t0, t1, zrows, arows,
              asum_sh, acc_sh):
    c = lax.axis_index("c")
    s = lax.axis_index("s")
    wid = s * NC + c

    pltpu.sync_copy(asum2_hbm.at[0, pl.ds(s * SLN, SLN)], t0)
    pltpu.sync_copy(asum2_hbm.at[1, pl.ds(s * SLN, SLN)], t1)
    for i in range(SLN // 16):
        sl = pl.ds(16 * i, 16)
        t0[sl] = t0[sl] + t1[sl]
    pltpu.sync_copy(t0, asum_sh.at[pl.ds(s * SLN, SLN)])

    for r in range(ZR):
        for i in range(FOUT // 16):
            zrows[r, pl.ds(16 * i, 16)] = jnp.zeros((16,), jnp.float32)
    for k in range(SLN // ZR):
        pltpu.sync_copy(zrows, acc_sh.at[pl.ds(s * SLN + k * ZR, ZR)])
    plsc.subcore_barrier()

    ebase = wid * EPW

    def chunk(i, carry):
        b = ebase + i * CH
        pltpu.sync_copy(src_hbm.at[pl.ds(b, CH)], srcb)
        pltpu.sync_copy(et_hbm.at[pl.ds(b, CH)], etb)
        pltpu.sync_copy(dst_hbm.at[pl.ds(b, CH)], dstb)
        pltpu.sync_copy(norm_hbm.at[pl.ds(b, CH)], normb)
        pltpu.sync_copy(evals_hbm.at[pl.ds(b, CH)], evb)
        for j in range(CH // 16):
            sl = pl.ds(16 * j, 16)
            gidxb[sl] = etb[sl] * NPAD + srcb[sl]
        pltpu.sync_copy(hp_hbm.at[gidxb], arows)
        pltpu.sync_copy(asum_sh.at[srcb], coefb)
        for j in range(CH // 16):
            sl = pl.ds(16 * j, 16)
            coefb[sl] = evb[sl] * normb[sl] / coefb[sl]

        def scale16(g, carry2):
            base = g * 16
            cvec = coefb[pl.ds(base, 16)]
            for lane in range(16):
                cb = jnp.full((16,), cvec[lane], jnp.float32)
                e = base + lane
                for j in range(FOUT // 16):
                    sl = pl.ds(16 * j, 16)
                    arows[e, sl] = arows[e, sl] * cb
            return carry2

        lax.fori_loop(0, CH // 16, scale16, 0)
        pltpu.sync_copy(arows, acc_sh.at[dstb], add=True)
        return carry

    lax.fori_loop(0, NCH, chunk, 0)

    plsc.subcore_barrier()
    pltpu.sync_copy(acc_sh.at[pl.ds(s * SLN, SLN)],
                    outp_hbm.at[c, pl.ds(s * SLN, SLN)])


_agg = pl.kernel(
    _agg_body,
    out_type=jax.ShapeDtypeStruct((NC, NPAD, FOUT), jnp.float32),
    mesh=plsc.VectorSubcoreMesh(core_axis_name="c", subcore_axis_name="s"),
    scratch_types=[
        pltpu.VMEM((CH,), jnp.int32),
        pltpu.VMEM((CH,), jnp.int32),
        pltpu.VMEM((CH,), jnp.int32),
        pltpu.VMEM((CH,), jnp.int32),
        pltpu.VMEM((CH,), jnp.float32),
        pltpu.VMEM((CH,), jnp.float32),
        pltpu.VMEM((CH,), jnp.float32),
        pltpu.VMEM((SLN,), jnp.float32),
        pltpu.VMEM((SLN,), jnp.float32),
        pltpu.VMEM((ZR, FOUT), jnp.float32),
        pltpu.VMEM((CH, FOUT), jnp.float32),
        pltpu.VMEM_SHARED((NPAD,), jnp.float32),
        pltpu.VMEM_SHARED((NPAD, FOUT), jnp.float32),
    ],
)


def _add_body(a_ref, b_ref, o_ref):
    o_ref[...] = a_ref[...] + b_ref[...]


_addk = pl.pallas_call(
    _add_body,
    grid=(NPAD // 1024,),
    in_specs=[pl.BlockSpec((1024, FOUT), lambda i: (i, 0)),
              pl.BlockSpec((1024, FOUT), lambda i: (i, 0))],
    out_specs=pl.BlockSpec((1024, FOUT), lambda i: (i, 0)),
    out_shape=jax.ShapeDtypeStruct((NPAD, FOUT), jnp.float32),
)


def kernel(x, edge_index, edge_type, edge_norm, weight, a_weight, w_comp, w_comp_a):
    xp = jnp.zeros((NPAD, FIN), jnp.float32).at[:N].set(x)
    src = edge_index[0]
    dst = edge_index[1]
    npad_e = EPAD - E
    srcp = jnp.concatenate([src, jnp.full((npad_e,), N, jnp.int32)])
    dstp = jnp.concatenate([dst, jnp.full((npad_e,), N, jnp.int32)])
    etp = jnp.concatenate([edge_type, jnp.zeros((npad_e,), jnp.int32)])
    normp = jnp.concatenate([edge_norm[:, 0], jnp.zeros((npad_e,), jnp.float32)])
    aw2 = a_weight[:, :, 0]

    hproj, eproj = _proj(xp, weight, aw2, w_comp, w_comp_a)
    hp_flat = hproj.reshape(R * NPAD, FOUT)
    ep_flat = eproj.reshape(NPAD * R)

    evals, asum2 = _atten(ep_flat, srcp, etp)
    outp = _agg(hp_flat, srcp, etp, dstp, normp, evals, asum2)
    out = _addk(outp[0], outp[1])
    return out[:N]
